# Initial kernel scaffold; baseline (speedup 1.0000x reference)
import functools

import jax
import jax.numpy as jnp
from jax import lax
from jax.experimental import pallas as pl
from jax.experimental.pallas import tpu as pltpu
from jax.experimental.pallas import tpu_sc as plsc

N_NODES = 10000
D = 128
N_EDGES = 320000

NC = 2
NS = 16
NW = NC * NS

CHUNK = 128
EDGES_PER_TILE = -(-N_EDGES // (NW * CHUNK)) * CHUNK
E_PAD = EDGES_PER_TILE * NW
CHUNKS_PER_TILE = EDGES_PER_TILE // CHUNK
PAD_ROW = N_NODES
N_SPMEM = N_NODES + 16
ROWS_PER_TILE = N_NODES // NS


def _sc_body(x_hbm, col_hbm, row_hbm, out_hbm, acc, col_v, row_v, buf, sem0, sem1):
    c = lax.axis_index("c")
    s = lax.axis_index("s")
    wid = c * NS + s

    idx_base = wid * CHUNKS_PER_TILE
    cp_c = pltpu.async_copy(col_hbm.at[pl.ds(idx_base, CHUNKS_PER_TILE)], col_v, sem0)
    cp_r = pltpu.async_copy(row_hbm.at[pl.ds(idx_base, CHUNKS_PER_TILE)], row_v, sem1)

    r0 = s * ROWS_PER_TILE
    pltpu.sync_copy(x_hbm.at[pl.ds(r0, ROWS_PER_TILE)], acc.at[pl.ds(r0, ROWS_PER_TILE)])
    cp_c.wait()
    cp_r.wait()
    plsc.subcore_barrier()

    @pl.loop(0, CHUNKS_PER_TILE, step=2)
    def _edge_loop(j):
        g0 = pltpu.async_copy(x_hbm.at[col_v.at[j]], buf.at[0], sem0)
        g1 = pltpu.async_copy(x_hbm.at[col_v.at[j + 1]], buf.at[1], sem1)
        g0.wait()
        pltpu.sync_copy(buf.at[0], acc.at[row_v.at[j]], add=True)
        g1.wait()
        pltpu.sync_copy(buf.at[1], acc.at[row_v.at[j + 1]], add=True)

    plsc.subcore_barrier()

    pltpu.sync_copy(acc.at[pl.ds(r0, ROWS_PER_TILE)], out_hbm.at[c, pl.ds(r0, ROWS_PER_TILE)])


_sc_agg = pl.kernel(
    _sc_body,
    out_type=jax.ShapeDtypeStruct((NC, N_NODES, D), jnp.float32),
    mesh=plsc.VectorSubcoreMesh(core_axis_name="c", subcore_axis_name="s"),
    scratch_types=[
        pltpu.VMEM_SHARED((N_SPMEM, D), jnp.float32),
        pltpu.VMEM((CHUNKS_PER_TILE, CHUNK), jnp.int32),
        pltpu.VMEM((CHUNKS_PER_TILE, CHUNK), jnp.int32),
        pltpu.VMEM((2, CHUNK, D), jnp.float32),
        pltpu.SemaphoreType.DMA,
        pltpu.SemaphoreType.DMA,
    ],
)


def _mlp_body(x_ref, a_ref, w1_ref, b1_ref, w2_ref, b2_ref, o_ref):
    s = a_ref[0] + a_ref[1] - x_ref[...]
    h = jnp.dot(s, w1_ref[...], preferred_element_type=jnp.float32) + b1_ref[...]
    h = jnp.maximum(h, 0.0)
    o_ref[...] = jnp.dot(h, w2_ref[...], preferred_element_type=jnp.float32) + b2_ref[...]


_MLP_BLOCK = 2000


def _mlp(x, a, W1, b1, W2, b2):
    grid = (N_NODES // _MLP_BLOCK,)
    return pl.pallas_call(
        _mlp_body,
        grid=grid,
        in_specs=[
            pl.BlockSpec((_MLP_BLOCK, D), lambda i: (i, 0)),
            pl.BlockSpec((NC, _MLP_BLOCK, D), lambda i: (0, i, 0)),
            pl.BlockSpec((D, D), lambda i: (0, 0)),
            pl.BlockSpec((1, D), lambda i: (0, 0)),
            pl.BlockSpec((D, D), lambda i: (0, 0)),
            pl.BlockSpec((1, D), lambda i: (0, 0)),
        ],
        out_specs=pl.BlockSpec((_MLP_BLOCK, D), lambda i: (i, 0)),
        out_shape=jax.ShapeDtypeStruct((N_NODES, D), jnp.float32),
    )(x, a, W1, b1, W2, b2)


@jax.jit
def kernel(x, edge_index, W1, b1, W2, b2):
    ei = edge_index.astype(jnp.int32)
    pad = E_PAD - N_EDGES
    col = jnp.concatenate([ei[1], jnp.zeros((pad,), jnp.int32)]).reshape(-1, CHUNK)
    row = jnp.concatenate([ei[0], jnp.full((pad,), PAD_ROW, jnp.int32)]).reshape(-1, CHUNK)
    a = _sc_agg(x, col, row)
    return _mlp(x, a, W1, b1.reshape(1, D), W2, b2.reshape(1, D))


# speedup vs baseline: 3.5209x; 3.5209x over previous
import functools

import jax
import jax.numpy as jnp
from jax import lax
from jax.experimental import pallas as pl
from jax.experimental.pallas import tpu as pltpu
from jax.experimental.pallas import tpu_sc as plsc

N_NODES = 10000
D = 128
N_EDGES = 320000

NC = 2
NS = 16
NW = NC * NS

CHUNK = 128
CHUNKS_PER_TILE = -(-N_EDGES // (NW * CHUNK * 8)) * 8
EDGES_PER_TILE = CHUNKS_PER_TILE * CHUNK
E_PAD = EDGES_PER_TILE * NW
PAD_ROW = N_NODES
N_SPMEM = N_NODES + 16
ROWS_PER_TILE = (N_NODES // NS) // 8 * 8
TAIL_BASE = ROWS_PER_TILE * NS
TAIL_ROWS = N_NODES - TAIL_BASE


SUPER = 16
NSUPER = CHUNKS_PER_TILE // SUPER


def _sc_body(x_hbm, col_hbm, row_hbm, out_hbm, acc, col_v, row_v, buf,
             sem_g0, sem_g1, sem_i0, sem_i1):
    c = lax.axis_index("c")
    s = lax.axis_index("s")
    wid = c * NS + s
    idx_base = wid * CHUNKS_PER_TILE
    isems = (sem_i0, sem_i1)

    def stage_indices(t, p):
        cc = pltpu.async_copy(
            col_hbm.at[pl.ds(idx_base + t * SUPER, SUPER)], col_v.at[p], isems[p])
        cr = pltpu.async_copy(
            row_hbm.at[pl.ds(idx_base + t * SUPER, SUPER)], row_v.at[p], isems[p])
        return cc, cr

    pend = stage_indices(0, 0)

    r0 = s * ROWS_PER_TILE
    pltpu.sync_copy(x_hbm.at[pl.ds(r0, ROWS_PER_TILE)], acc.at[pl.ds(r0, ROWS_PER_TILE)])

    @pl.when(s == 0)
    def _init_tail():
        pltpu.sync_copy(x_hbm.at[pl.ds(TAIL_BASE, TAIL_ROWS)],
                        acc.at[pl.ds(TAIL_BASE, TAIL_ROWS)])

    plsc.subcore_barrier()

    for t in range(NSUPER):
        p = t & 1
        pend[0].wait()
        pend[1].wait()
        if t + 1 < NSUPER:
            pend = stage_indices(t + 1, 1 - p)

        @pl.loop(0, SUPER, step=2)
        def _edge_loop(j):
            g0 = pltpu.async_copy(x_hbm.at[col_v.at[p, j]], buf.at[0], sem_g0)
            g1 = pltpu.async_copy(x_hbm.at[col_v.at[p, j + 1]], buf.at[1], sem_g1)
            g0.wait()
            pltpu.sync_copy(buf.at[0], acc.at[row_v.at[p, j]], add=True)
            g1.wait()
            pltpu.sync_copy(buf.at[1], acc.at[row_v.at[p, j + 1]], add=True)

    plsc.subcore_barrier()

    pltpu.sync_copy(acc.at[pl.ds(r0, ROWS_PER_TILE)], out_hbm.at[c, pl.ds(r0, ROWS_PER_TILE)])

    @pl.when(s == 0)
    def _out_tail():
        pltpu.sync_copy(acc.at[pl.ds(TAIL_BASE, TAIL_ROWS)],
                        out_hbm.at[c, pl.ds(TAIL_BASE, TAIL_ROWS)])


_sc_agg = pl.kernel(
    _sc_body,
    out_type=jax.ShapeDtypeStruct((NC, N_NODES, D), jnp.float32),
    mesh=plsc.VectorSubcoreMesh(core_axis_name="c", subcore_axis_name="s"),
    scratch_types=[
        pltpu.VMEM_SHARED((N_SPMEM, D), jnp.float32),
        pltpu.VMEM((2, SUPER, CHUNK), jnp.int32),
        pltpu.VMEM((2, SUPER, CHUNK), jnp.int32),
        pltpu.VMEM((2, CHUNK, D), jnp.float32),
        pltpu.SemaphoreType.DMA,
        pltpu.SemaphoreType.DMA,
        pltpu.SemaphoreType.DMA,
        pltpu.SemaphoreType.DMA,
    ],
)


def _mlp_body(x_ref, a_ref, w1_ref, b1_ref, w2_ref, b2_ref, o_ref):
    s = a_ref[0] + a_ref[1] - x_ref[...]
    h = jnp.dot(s, w1_ref[...], preferred_element_type=jnp.float32) + b1_ref[...]
    h = jnp.maximum(h, 0.0)
    o_ref[...] = jnp.dot(h, w2_ref[...], preferred_element_type=jnp.float32) + b2_ref[...]


_MLP_BLOCK = 2000


def _mlp(x, a, W1, b1, W2, b2):
    grid = (N_NODES // _MLP_BLOCK,)
    return pl.pallas_call(
        _mlp_body,
        grid=grid,
        in_specs=[
            pl.BlockSpec((_MLP_BLOCK, D), lambda i: (i, 0)),
            pl.BlockSpec((NC, _MLP_BLOCK, D), lambda i: (0, i, 0)),
            pl.BlockSpec((D, D), lambda i: (0, 0)),
            pl.BlockSpec((1, D), lambda i: (0, 0)),
            pl.BlockSpec((D, D), lambda i: (0, 0)),
            pl.BlockSpec((1, D), lambda i: (0, 0)),
        ],
        out_specs=pl.BlockSpec((_MLP_BLOCK, D), lambda i: (i, 0)),
        out_shape=jax.ShapeDtypeStruct((N_NODES, D), jnp.float32),
    )(x, a, W1, b1, W2, b2)


@jax.jit
def kernel(x, edge_index, W1, b1, W2, b2):
    ei = edge_index.astype(jnp.int32)
    pad = E_PAD - N_EDGES
    col = jnp.concatenate([ei[1], jnp.zeros((pad,), jnp.int32)]).reshape(-1, CHUNK)
    row = jnp.concatenate([ei[0], jnp.full((pad,), PAD_ROW, jnp.int32)]).reshape(-1, CHUNK)
    a = _sc_agg(x, col, row)
    return _mlp(x, a, W1, b1.reshape(1, D), W2, b2.reshape(1, D))


# speedup vs baseline: 4.2587x; 1.2095x over previous
import jax
import jax.numpy as jnp
from jax import lax
from jax.experimental import pallas as pl
from jax.experimental.pallas import tpu as pltpu
from jax.experimental.pallas import tpu_sc as plsc

N_NODES = 10000
D = 128
N_EDGES = 320000

NC = 2
NS = 16

CHUNK = 128
CH0 = 128
CH1 = 32
E_PAD = NS * (CH0 + CH1) * CHUNK
PAD_ROW = N_NODES
N_SPMEM = N_NODES + 16
ROWS_PER_TILE = (N_NODES // NS) // 8 * 8
TAIL_BASE = ROWS_PER_TILE * NS
TAIL_ROWS = N_NODES - TAIL_BASE

SUPER = 16


def _sc_body(x_hbm, col_hbm, row_hbm, out_hbm, acc, col_v, row_v, buf,
             sem_g0, sem_g1, sem_s0, sem_s1, sem_i0, sem_i1):
    c = lax.axis_index("c")
    s = lax.axis_index("s")
    isems = (sem_i0, sem_i1)

    def stage_indices(idx_base, t, p):
        cc = pltpu.async_copy(
            col_hbm.at[pl.ds(idx_base + t * SUPER, SUPER)], col_v.at[p], isems[p])
        cr = pltpu.async_copy(
            row_hbm.at[pl.ds(idx_base + t * SUPER, SUPER)], row_v.at[p], isems[p])
        return cc, cr

    def run_edges(idx_base, nchunks):
        pend = stage_indices(idx_base, 0, 0)
        plsc.subcore_barrier()
        nsuper = nchunks // SUPER
        for t in range(nsuper):
            p = t & 1
            pend[0].wait()
            pend[1].wait()
            if t + 1 < nsuper:
                pend = stage_indices(idx_base, t + 1, 1 - p)

            @pl.loop(0, SUPER, step=2)
            def _edge_loop(j):
                g0 = pltpu.async_copy(x_hbm.at[col_v.at[p, j]], buf.at[0], sem_g0)
                g1 = pltpu.async_copy(x_hbm.at[col_v.at[p, j + 1]], buf.at[1], sem_g1)
                g0.wait()
                s0 = pltpu.async_copy(buf.at[0], acc.at[row_v.at[p, j]], sem_s0,
                                      add=True)
                g1.wait()
                s1 = pltpu.async_copy(buf.at[1], acc.at[row_v.at[p, j + 1]], sem_s1,
                                      add=True)
                s0.wait()
                s1.wait()

    r0 = s * ROWS_PER_TILE

    @pl.when(c == 0)
    def _core0():
        pltpu.sync_copy(x_hbm.at[pl.ds(r0, ROWS_PER_TILE)],
                        acc.at[pl.ds(r0, ROWS_PER_TILE)])

        @pl.when(s == 0)
        def _init_tail():
            pltpu.sync_copy(x_hbm.at[pl.ds(TAIL_BASE, TAIL_ROWS)],
                            acc.at[pl.ds(TAIL_BASE, TAIL_ROWS)])

        run_edges(s * CH0, CH0)

    @pl.when(c == 1)
    def _core1():
        @pl.loop(0, CHUNK)
        def _zrow(r):
            @pl.loop(0, D // 16)
            def _zcol(k):
                buf[0, r, pl.ds(k * 16, 16)] = jnp.zeros((16,), jnp.float32)

        for q in range(4):
            pltpu.sync_copy(buf.at[0], acc.at[pl.ds(r0 + q * CHUNK, CHUNK)])
        pltpu.sync_copy(buf.at[0, pl.ds(0, ROWS_PER_TILE - 4 * CHUNK)],
                        acc.at[pl.ds(r0 + 4 * CHUNK, ROWS_PER_TILE - 4 * CHUNK)])

        @pl.when(s == 0)
        def _init_tail():
            pltpu.sync_copy(buf.at[0, pl.ds(0, TAIL_ROWS)],
                            acc.at[pl.ds(TAIL_BASE, TAIL_ROWS)])

        run_edges(NS * CH0 + s * CH1, CH1)

    plsc.subcore_barrier()

    pltpu.sync_copy(acc.at[pl.ds(r0, ROWS_PER_TILE)], out_hbm.at[c, pl.ds(r0, ROWS_PER_TILE)])

    @pl.when(s == 0)
    def _out_tail():
        pltpu.sync_copy(acc.at[pl.ds(TAIL_BASE, TAIL_ROWS)],
                        out_hbm.at[c, pl.ds(TAIL_BASE, TAIL_ROWS)])


_sc_agg = pl.kernel(
    _sc_body,
    out_type=jax.ShapeDtypeStruct((NC, N_NODES, D), jnp.float32),
    mesh=plsc.VectorSubcoreMesh(core_axis_name="c", subcore_axis_name="s"),
    scratch_types=[
        pltpu.VMEM_SHARED((N_SPMEM, D), jnp.float32),
        pltpu.VMEM((2, SUPER, CHUNK), jnp.int32),
        pltpu.VMEM((2, SUPER, CHUNK), jnp.int32),
        pltpu.VMEM((2, CHUNK, D), jnp.float32),
        pltpu.SemaphoreType.DMA,
        pltpu.SemaphoreType.DMA,
        pltpu.SemaphoreType.DMA,
        pltpu.SemaphoreType.DMA,
        pltpu.SemaphoreType.DMA,
        pltpu.SemaphoreType.DMA,
    ],
)


def _mlp_body(a_ref, w1_ref, b1_ref, w2_ref, b2_ref, o_ref):
    s = a_ref[0] + a_ref[1]
    h = jnp.dot(s, w1_ref[...], preferred_element_type=jnp.float32) + b1_ref[...]
    h = jnp.maximum(h, 0.0)
    o_ref[...] = jnp.dot(h, w2_ref[...], preferred_element_type=jnp.float32) + b2_ref[...]


_MLP_BLOCK = 2000


def _mlp(a, W1, b1, W2, b2):
    grid = (N_NODES // _MLP_BLOCK,)
    return pl.pallas_call(
        _mlp_body,
        grid=grid,
        in_specs=[
            pl.BlockSpec((NC, _MLP_BLOCK, D), lambda i: (0, i, 0)),
            pl.BlockSpec((D, D), lambda i: (0, 0)),
            pl.BlockSpec((1, D), lambda i: (0, 0)),
            pl.BlockSpec((D, D), lambda i: (0, 0)),
            pl.BlockSpec((1, D), lambda i: (0, 0)),
        ],
        out_specs=pl.BlockSpec((_MLP_BLOCK, D), lambda i: (i, 0)),
        out_shape=jax.ShapeDtypeStruct((N_NODES, D), jnp.float32),
    )(a, W1, b1, W2, b2)


@jax.jit
def kernel(x, edge_index, W1, b1, W2, b2):
    ei = edge_index.astype(jnp.int32)
    pad = E_PAD - N_EDGES
    col = jnp.concatenate([ei[1], jnp.zeros((pad,), jnp.int32)]).reshape(-1, CHUNK)
    row = jnp.concatenate([ei[0], jnp.full((pad,), PAD_ROW, jnp.int32)]).reshape(-1, CHUNK)
    a = _sc_agg(x, col, row)
    return _mlp(a, W1, b1.reshape(1, D), W2, b2.reshape(1, D))
